# baseline (speedup 1.0000x reference)
import jax
import jax.numpy as jnp
from jax import lax
from jax.experimental import pallas as pl
from jax.experimental.pallas import tpu as pltpu, tpu_sc as plsc

D = 1024
S = 8192
B = 4
NW = 32
ROWS_PER_W = S // NW
CHUNK = 8
N_CHUNKS = ROWS_PER_W // CHUNK
NXB = 3
LEAD = 2
CSL = D // 16


def _sc_body(x_hbm, pos_hbm, out_hbm, x_v, pos_v, sem_xl, sem_pl, sem_st):
    cid = lax.axis_index("c")
    sid = lax.axis_index("s")
    wid = sid * 2 + cid
    row_base = wid * ROWS_PER_W

    def xslice(j):
        return x_hbm.at[:, pl.ds(row_base + j * CHUNK, CHUNK), :]

    def oslice(j):
        return out_hbm.at[:, pl.ds(row_base + j * CHUNK, CHUNK), :]

    def start_xload(j):
        pltpu.async_copy(xslice(j), x_v.at[j % NXB], sem_xl)

    def start_pload(t):
        pltpu.async_copy(pos_hbm.at[pl.ds(row_base + t * CHUNK, CHUNK), :],
                         pos_v.at[t % 2], sem_pl)

    def wait_xload(slot):
        pltpu.make_async_copy(x_hbm.at[:, pl.ds(0, CHUNK), :], x_v.at[slot],
                              sem_xl).wait()

    def wait_pload(slot):
        pltpu.make_async_copy(pos_hbm.at[pl.ds(0, CHUNK), :], pos_v.at[slot],
                              sem_pl).wait()

    def wait_store(slot):
        pltpu.make_async_copy(x_v.at[slot], out_hbm.at[:, pl.ds(0, CHUNK), :],
                              sem_st).wait()

    start_pload(0)
    start_pload(1)
    start_xload(0)
    start_xload(1)

    for j in range(N_CHUNKS):
        s = j % NXB
        ps = j % 2

        wait_pload(ps)
        wait_xload(s)

        xv = x_v.at[s]
        pv = pos_v.at[ps]

        def add_loop(i, _):
            b = i // (CHUNK * CSL)
            r = (i // CSL) % CHUNK
            c = (i % CSL) * 16
            sl = pl.ds(c, 16)
            xv[b, r, sl] = xv[b, r, sl] + pv[r, sl]
            return 0

        lax.fori_loop(0, B * CHUNK * CSL, add_loop, 0, unroll=8)

        pltpu.async_copy(xv, oslice(j), sem_st)

        if j + 2 < N_CHUNKS:
            start_pload(j + 2)

        if j + LEAD < N_CHUNKS:
            if j >= NXB - LEAD:
                wait_store((j + LEAD) % NXB)
            start_xload(j + LEAD)

    for j in range(N_CHUNKS - NXB, N_CHUNKS):
        wait_store(j % NXB)


_sc_call = pl.kernel(
    _sc_body,
    out_type=jax.ShapeDtypeStruct((B, S, D), jnp.float32),
    mesh=plsc.VectorSubcoreMesh(core_axis_name="c", subcore_axis_name="s"),
    scratch_types=[
        pltpu.VMEM((NXB, B, CHUNK, D), jnp.float32),
        pltpu.VMEM((2, CHUNK, D), jnp.float32),
        pltpu.SemaphoreType.DMA,
        pltpu.SemaphoreType.DMA,
        pltpu.SemaphoreType.DMA,
    ],
    compiler_params=pltpu.CompilerParams(use_tc_tiling_on_sc=True),
)


def kernel(x, pos_table):
    return _sc_call(x, pos_table)
